# baseline (speedup 1.0000x reference)
import functools

import jax
import jax.numpy as jnp
from jax import lax
from jax.experimental import pallas as pl
from jax.experimental.pallas import tpu as pltpu
from jax.experimental.pallas import tpu_sc as plsc

_B, _F, _V, _E, _NUM = 4096, 26, 100000, 64, 13
_D = _F * _E + _E
_NC, _NS = 2, 16
_NW = _NC * _NS
_RPW = _B * _F // _NW
_J = _RPW // 128
_JP = 32


def _make_sc_gather():
    mesh = plsc.VectorSubcoreMesh(
        core_axis_name="c", subcore_axis_name="s",
        num_cores=_NC, num_subcores=_NS)

    @functools.partial(
        pl.kernel,
        out_type=jax.ShapeDtypeStruct((_NW, _J, 128, 128), jnp.float32),
        mesh=mesh,
        scratch_types=[
            pltpu.VMEM((_JP, 128), jnp.int32),
            pltpu.VMEM((128, 128), jnp.float32),
            pltpu.SemaphoreType.DMA,
        ],
    )
    def sc_gather(idx_hbm, tab_hbm, out_hbm, idx_v, rows_v, sem):
        wid = lax.axis_index("s") * _NC + lax.axis_index("c")
        pltpu.sync_copy(idx_hbm.at[wid], idx_v)

        def body(j, carry):
            pltpu.async_copy(tab_hbm.at[idx_v.at[j]], rows_v, sem).wait()
            pltpu.sync_copy(rows_v, out_hbm.at[wid, j])
            return carry

        lax.fori_loop(0, _J, body, 0, unroll=False)

    return sc_gather


_SC_GATHER_CACHE = []


def _sc_gather_fn():
    if not _SC_GATHER_CACHE:
        _SC_GATHER_CACHE.append(_make_sc_gather())
    return _SC_GATHER_CACHE[0]


def _tc_dense_body(g_ref, sel_ref, xn_ref, wn_ref, bn_ref, wc_ref, wnt_ref,
                   bf_ref, out_ref):
    num_emb = lax.dot_general(
        xn_ref[...], wn_ref[...], (((1,), (1,)), ((), ())),
        preferred_element_type=jnp.float32) + bn_ref[...]
    o = bf_ref[...] + lax.dot_general(
        num_emb, wnt_ref[...], (((1,), (0,)), ((), ())),
        preferred_element_type=jnp.float32)
    g = g_ref[...]
    m = sel_ref[...][:, :, None]
    emb = g[:, :, :_E] * (1.0 - m) + g[:, :, _E:] * m
    for f in range(_F):
        o = o + lax.dot_general(
            emb[:, f, :], wc_ref[f], (((1,), (0,)), ((), ())),
            preferred_element_type=jnp.float32)
    out_ref[...] = o


def _tc_dense(g, sel, X_num, W_num, b_num2, Wc3, WnT, b_final2):
    BB = 512
    return pl.pallas_call(
        _tc_dense_body,
        grid=(_B // BB,),
        in_specs=[
            pl.BlockSpec((BB, _F, 128), lambda i: (i, 0, 0)),
            pl.BlockSpec((BB, _F), lambda i: (i, 0)),
            pl.BlockSpec((BB, _NUM), lambda i: (i, 0)),
            pl.BlockSpec((_E, _NUM), lambda i: (0, 0)),
            pl.BlockSpec((1, _E), lambda i: (0, 0)),
            pl.BlockSpec((_F, _E, _E), lambda i: (0, 0, 0)),
            pl.BlockSpec((_E, _E), lambda i: (0, 0)),
            pl.BlockSpec((1, _E), lambda i: (0, 0)),
        ],
        out_specs=pl.BlockSpec((BB, _E), lambda i: (i, 0)),
        out_shape=jax.ShapeDtypeStruct((_B, _E), jnp.float32),
    )(g, sel, X_num, W_num, b_num2, Wc3, WnT, b_final2)


def kernel(X_cat, X_num, tables, W_num, b_num, W_final, b_final):
    xc = X_cat.astype(jnp.int32)
    flat_row = xc + (jnp.arange(_F, dtype=jnp.int32) * _V)[None, :]
    pair_idx = (flat_row >> 1).reshape(_NW, _J, 128)
    pair_idx = jnp.pad(pair_idx, ((0, 0), (0, _JP - _J), (0, 0)))
    sel = (xc & 1).astype(jnp.float32)
    tab2 = tables.reshape(_F * _V // 2, 128)

    gathered = _sc_gather_fn()(pair_idx, tab2)
    g = gathered.reshape(_B, _F, 128)

    Wc3 = jnp.transpose(W_final[:, : _F * _E].reshape(_E, _F, _E), (1, 2, 0))
    WnT = W_final[:, _F * _E:].T
    return _tc_dense(g, sel, X_num, W_num, b_num.reshape(1, _E),
                     Wc3, WnT, b_final.reshape(1, _E))


# speedup vs baseline: 1.0379x; 1.0379x over previous
import functools

import jax
import jax.numpy as jnp
from jax import lax
from jax.experimental import pallas as pl
from jax.experimental.pallas import tpu as pltpu
from jax.experimental.pallas import tpu_sc as plsc

_B, _F, _V, _E, _NUM = 4096, 26, 100000, 64, 13
_D = _F * _E + _E
_NC, _NS = 2, 16
_NW = _NC * _NS
_BPW = _B // _NW
_JP = 32


def _make_sc_gather():
    mesh = plsc.VectorSubcoreMesh(
        core_axis_name="c", subcore_axis_name="s",
        num_cores=_NC, num_subcores=_NS)

    @functools.partial(
        pl.kernel,
        out_type=jax.ShapeDtypeStruct((_NW, _F, _BPW, _E), jnp.float32),
        mesh=mesh,
        scratch_types=[
            pltpu.VMEM((_JP, _BPW), jnp.int32),
            pltpu.VMEM((_BPW, _E), jnp.float32),
            pltpu.SemaphoreType.DMA,
        ],
        compiler_params=pltpu.CompilerParams(use_tc_tiling_on_sc=False),
    )
    def sc_gather(idx_hbm, tab_hbm, out_hbm, idx_v, rows_v, sem):
        wid = lax.axis_index("s") * _NC + lax.axis_index("c")
        pltpu.sync_copy(idx_hbm.at[wid], idx_v)

        def body(f, carry):
            pltpu.async_copy(tab_hbm.at[f].at[idx_v.at[f]], rows_v, sem).wait()
            pltpu.sync_copy(rows_v, out_hbm.at[wid, f])
            return carry

        lax.fori_loop(0, _F, body, 0, unroll=False)

    return sc_gather


_SC_GATHER_CACHE = []


def _sc_gather_fn():
    if not _SC_GATHER_CACHE:
        _SC_GATHER_CACHE.append(_make_sc_gather())
    return _SC_GATHER_CACHE[0]


def _tc_dense_body(g_ref, xn_ref, wn_ref, bn_ref, wc_ref, wnt_ref,
                   bf_ref, out_ref):
    num_emb = lax.dot_general(
        xn_ref[...], wn_ref[...], (((1,), (1,)), ((), ())),
        preferred_element_type=jnp.float32) + bn_ref[...]
    o = bf_ref[...] + lax.dot_general(
        num_emb, wnt_ref[...], (((1,), (0,)), ((), ())),
        preferred_element_type=jnp.float32)
    g = g_ref[...]
    for f in range(_F):
        o = o + lax.dot_general(
            g[0, f], wc_ref[f], (((1,), (0,)), ((), ())),
            preferred_element_type=jnp.float32)
    out_ref[...] = o


def _tc_dense(g, X_num, W_num, b_num2, Wc3, WnT, b_final2):
    return pl.pallas_call(
        _tc_dense_body,
        grid=(_NW,),
        in_specs=[
            pl.BlockSpec((1, _F, _BPW, _E), lambda i: (i, 0, 0, 0)),
            pl.BlockSpec((_BPW, _NUM), lambda i: (i, 0)),
            pl.BlockSpec((_E, _NUM), lambda i: (0, 0)),
            pl.BlockSpec((1, _E), lambda i: (0, 0)),
            pl.BlockSpec((_F, _E, _E), lambda i: (0, 0, 0)),
            pl.BlockSpec((_E, _E), lambda i: (0, 0)),
            pl.BlockSpec((1, _E), lambda i: (0, 0)),
        ],
        out_specs=pl.BlockSpec((_BPW, _E), lambda i: (i, 0)),
        out_shape=jax.ShapeDtypeStruct((_B, _E), jnp.float32),
    )(g, X_num, W_num, b_num2, Wc3, WnT, b_final2)


def kernel(X_cat, X_num, tables, W_num, b_num, W_final, b_final):
    xc = X_cat.astype(jnp.int32)
    idx = jnp.transpose(xc.reshape(_NW, _BPW, _F), (0, 2, 1))
    idx = jnp.pad(idx, ((0, 0), (0, _JP - _F), (0, 0)))

    gathered = _sc_gather_fn()(idx, tables)

    Wc3 = jnp.transpose(W_final[:, : _F * _E].reshape(_E, _F, _E), (1, 2, 0))
    WnT = W_final[:, _F * _E:].T
    return _tc_dense(gathered, X_num, W_num, b_num.reshape(1, _E),
                     Wc3, WnT, b_final.reshape(1, _E))
